# baseline (speedup 1.0000x reference)
import jax
import jax.numpy as jnp
from jax import lax
from jax.experimental import pallas as pl
from jax.experimental.pallas import tpu as pltpu
from jax.experimental.pallas import tpu_sc as plsc

N = 100000
NC, NS = 2, 16
CHUNK = 256
W = 8
IB = 2
G = 3
GD = 2
SUP = IB * W * CHUNK
R = 100352
ZROWS = R // NS
BN = 2000

_mesh = plsc.VectorSubcoreMesh(
    core_axis_name="c", subcore_axis_name="s", num_cores=NC, num_subcores=NS)


def _edge_loop(edges, tab, acc, idxb, rows, semi, semg, sems,
               blk_base, n_blk):
  M = IB * W

  @pl.loop(0, n_blk // IB)
  def _outer(i):
    b0 = blk_base + i * IB
    ld = [pltpu.async_copy(edges.at[b0 + sl], idxb.at[sl], semi)
          for sl in range(IB)]
    gl = [None] * M
    sc = [None] * M
    for sl in range(IB):
      ld[sl].wait()
      for t in range(W):
        g = sl * W + t
        if g >= G:
          sc[g - G].wait()
        gl[g] = pltpu.async_copy(tab.at[idxb.at[sl, t, 0]], rows.at[g % G],
                                 semg)
        if g >= GD:
          gl[g - GD].wait()
          psl, pt = divmod(g - GD, W)
          sc[g - GD] = pltpu.async_copy(rows.at[(g - GD) % G],
                                        acc.at[idxb.at[psl, pt, 1]],
                                        sems, add=True)
    for g in range(M - GD, M):
      gl[g].wait()
      psl, pt = divmod(g, W)
      sc[g] = pltpu.async_copy(rows.at[g % G], acc.at[idxb.at[psl, pt, 1]],
                               sems, add=True)
    for g in range(M - G, M):
      sc[g].wait()


def _make_scatter(F, total_blocks):
  def body(edges, ta, tb, zer, out, idxb, rows, acc, semi, semg, sems):
    c = lax.axis_index("c")
    s = lax.axis_index("s")
    sl = pl.ds(s * ZROWS, ZROWS)
    pltpu.sync_copy(zer, acc.at[sl])
    plsc.subcore_barrier()
    bpt = total_blocks // NS
    base = s * bpt
    pl.when(c == 0)(lambda: _edge_loop(
        edges, ta, acc, idxb, rows, semi, semg, sems, base, bpt))
    pl.when(c == 1)(lambda: _edge_loop(
        edges, tb, acc, idxb, rows, semi, semg, sems, base, bpt))
    plsc.subcore_barrier()
    pl.when(c == 0)(lambda: pltpu.sync_copy(acc.at[sl], out.at[0, sl]))
    pl.when(c == 1)(lambda: pltpu.sync_copy(acc.at[sl], out.at[1, sl]))

  return pl.kernel(
      body,
      out_type=jax.ShapeDtypeStruct((NC, R, F), jnp.float32),
      mesh=_mesh,
      compiler_params=pltpu.CompilerParams(use_tc_tiling_on_sc=False),
      scratch_types=[
          pltpu.VMEM((IB, W, 2, CHUNK), jnp.int32),
          pltpu.VMEM((G, CHUNK, F), jnp.float32),
          pltpu.VMEM_SHARED((R, F), jnp.float32),
          pltpu.SemaphoreType.DMA,
          pltpu.SemaphoreType.DMA,
          pltpu.SemaphoreType.DMA,
      ],
  )


def _make_deg(total_blocks):
  def body(edges, zer, out, idxb, ones_v, acc, semi, sems):
    c = lax.axis_index("c")
    s = lax.axis_index("s")
    sl = pl.ds(s * ZROWS, ZROWS)
    for i in range(CHUNK // 16):
      ones_v[pl.ds(i * 16, 16)] = jnp.ones((16,), jnp.float32)
    pltpu.sync_copy(zer, acc.at[sl])
    plsc.subcore_barrier()
    bpt = total_blocks // (NC * NS)
    base = (s * NC + c) * bpt

    @pl.loop(0, bpt // IB)
    def _outer(i):
      b0 = base + i * IB
      ld = [pltpu.async_copy(edges.at[b0 + sl_], idxb.at[sl_], semi)
            for sl_ in range(IB)]
      sc = []
      for sl_ in range(IB):
        ld[sl_].wait()
        for t in range(W):
          sc.append(pltpu.async_copy(ones_v, acc.at[idxb.at[sl_, t, 1]],
                                     sems, add=True))
      for s_ in sc:
        s_.wait()

    plsc.subcore_barrier()
    pl.when(c == 0)(lambda: pltpu.sync_copy(acc.at[sl], out.at[0, sl]))
    pl.when(c == 1)(lambda: pltpu.sync_copy(acc.at[sl], out.at[1, sl]))

  return pl.kernel(
      body,
      out_type=jax.ShapeDtypeStruct((NC, R), jnp.float32),
      mesh=_mesh,
      compiler_params=pltpu.CompilerParams(use_tc_tiling_on_sc=False),
      scratch_types=[
          pltpu.VMEM((IB, W, 2, CHUNK), jnp.int32),
          pltpu.VMEM((CHUNK,), jnp.float32),
          pltpu.VMEM_SHARED((R,), jnp.float32),
          pltpu.SemaphoreType.DMA,
          pltpu.SemaphoreType.DMA,
      ],
  )


def _mm1_body(x_ref, w_ref, o_ref):
  o_ref[...] = jnp.dot(x_ref[...], w_ref[...],
                       preferred_element_type=jnp.float32)


def _scale1_body(h_ref, dp0_ref, dp1_ref, hsa_ref, hsb_ref, dis_ref):
  dis = lax.rsqrt(dp0_ref[...] + dp1_ref[...] + 1.0)
  hs = h_ref[...] * dis
  hsa_ref[...] = hs[:, :16]
  hsb_ref[...] = hs[:, 16:]
  dis_ref[...] = dis


def _mid_body(a0_ref, a1_ref, hsa_ref, hsb_ref, dis_ref, w2_ref, b1_ref,
              hs2a_ref, hs2b_ref):
  dis = dis_ref[...]
  b1 = b1_ref[...]
  r0 = jnp.maximum((a0_ref[...] + hsa_ref[...]) * dis + b1[:, :16], 0.0)
  r1 = jnp.maximum((a1_ref[...] + hsb_ref[...]) * dis + b1[:, 16:], 0.0)
  w2 = w2_ref[...]
  h2 = (jnp.dot(r0, w2[:16, :], preferred_element_type=jnp.float32)
        + jnp.dot(r1, w2[16:, :], preferred_element_type=jnp.float32))
  hs2 = h2 * dis
  zpad = jnp.zeros((hs2.shape[0], 6), jnp.float32)
  hs2a_ref[...] = jnp.concatenate([hs2[:, :10], zpad], axis=-1)
  hs2b_ref[...] = jnp.concatenate([hs2[:, 10:], zpad], axis=-1)


def _post_body(a0_ref, a1_ref, hs2a_ref, hs2b_ref, dis_ref, b2_ref, o_ref):
  dis = dis_ref[...]
  b2 = b2_ref[...]
  v0 = (a0_ref[...] + hs2a_ref[...])[:, :10] * dis + b2[:, :10]
  v1 = (a1_ref[...] + hs2b_ref[...])[:, :10] * dis + b2[:, 10:]
  o_ref[...] = jnp.concatenate([v0, v1], axis=-1)


def _row_block(F):
  return pl.BlockSpec((BN, F), lambda i: (i, 0))


def _full_block(shape):
  return pl.BlockSpec(shape, lambda i: (0, 0))


def kernel(x, edge_index, W1, b1, W2, b2):
  x = x.astype(jnp.float32)
  ei = edge_index.astype(jnp.int32)
  E = ei.shape[1]
  group = NC * NS * SUP
  E_pad = ((E + group - 1) // group) * group
  pad = E_pad - E
  src = jnp.concatenate([ei[0], jnp.zeros((pad,), jnp.int32)])
  dst = jnp.concatenate([ei[1], jnp.full((pad,), N, jnp.int32)])
  edges = jnp.stack([src.reshape(-1, W, CHUNK), dst.reshape(-1, W, CHUNK)],
                    axis=2)
  total_blocks = E_pad // (W * CHUNK)
  z16 = jnp.zeros((ZROWS, 16), jnp.float32)
  zflat = jnp.zeros((ZROWS,), jnp.float32)

  grid = (N // BN,)

  degp = _make_deg(total_blocks)(edges, zflat)
  h1 = pl.pallas_call(
      _mm1_body, grid=grid,
      in_specs=[_row_block(20), _full_block((20, 32))],
      out_specs=_row_block(32),
      out_shape=jax.ShapeDtypeStruct((N, 32), jnp.float32))(x, W1)

  dp0 = degp[0, :N].reshape(N, 1)
  dp1 = degp[1, :N].reshape(N, 1)
  hsa, hsb, dis = pl.pallas_call(
      _scale1_body, grid=grid,
      in_specs=[_row_block(32), _row_block(1), _row_block(1)],
      out_specs=[_row_block(16), _row_block(16), _row_block(1)],
      out_shape=[jax.ShapeDtypeStruct((N, 16), jnp.float32),
                 jax.ShapeDtypeStruct((N, 16), jnp.float32),
                 jax.ShapeDtypeStruct((N, 1), jnp.float32)])(h1, dp0, dp1)

  agg1 = _make_scatter(16, total_blocks)(edges, hsa, hsb, z16)
  a10 = agg1[0, :N]
  a11 = agg1[1, :N]

  hs2a, hs2b = pl.pallas_call(
      _mid_body, grid=grid,
      in_specs=[_row_block(16), _row_block(16), _row_block(16),
                _row_block(16), _row_block(1), _full_block((32, 20)),
                _full_block((1, 32))],
      out_specs=[_row_block(16), _row_block(16)],
      out_shape=[jax.ShapeDtypeStruct((N, 16), jnp.float32),
                 jax.ShapeDtypeStruct((N, 16), jnp.float32)])(
          a10, a11, hsa, hsb, dis, W2, b1.reshape(1, 32))

  agg2 = _make_scatter(16, total_blocks)(edges, hs2a, hs2b, z16)
  a20 = agg2[0, :N]
  a21 = agg2[1, :N]

  out = pl.pallas_call(
      _post_body, grid=grid,
      in_specs=[_row_block(16), _row_block(16), _row_block(16),
                _row_block(16), _row_block(1), _full_block((1, 20))],
      out_specs=_row_block(20),
      out_shape=jax.ShapeDtypeStruct((N, 20), jnp.float32))(
          a20, a21, hs2a, hs2b, dis, b2.reshape(1, 20))
  return out


# speedup vs baseline: 1.4301x; 1.4301x over previous
import jax
import jax.numpy as jnp
from jax import lax
from jax.experimental import pallas as pl
from jax.experimental.pallas import tpu as pltpu
from jax.experimental.pallas import tpu_sc as plsc

N = 100000
NC, NS = 2, 16
CHUNK = 128
W = 8
IB = 2
G = 8
GD = 6
SUP = IB * W * CHUNK
R = 100352
ZROWS = R // NS
BN = 2000

_mesh = plsc.VectorSubcoreMesh(
    core_axis_name="c", subcore_axis_name="s", num_cores=NC, num_subcores=NS)


def _edge_loop(edges, tab, acc, idxb, rows, semi, semg, sems,
               blk_base, n_blk):
  M = IB * W

  @pl.loop(0, n_blk // IB)
  def _outer(i):
    b0 = blk_base + i * IB
    ld = [pltpu.async_copy(edges.at[b0 + sl], idxb.at[sl], semi)
          for sl in range(IB)]
    gl = [None] * M
    sc = [None] * M
    for sl in range(IB):
      ld[sl].wait()
      for t in range(W):
        g = sl * W + t
        if g >= G:
          sc[g - G].wait()
        gl[g] = pltpu.async_copy(tab.at[idxb.at[sl, t, 0]], rows.at[g % G],
                                 semg)
        if g >= GD:
          gl[g - GD].wait()
          psl, pt = divmod(g - GD, W)
          sc[g - GD] = pltpu.async_copy(rows.at[(g - GD) % G],
                                        acc.at[idxb.at[psl, pt, 1]],
                                        sems, add=True)
    for g in range(M - GD, M):
      gl[g].wait()
      psl, pt = divmod(g, W)
      sc[g] = pltpu.async_copy(rows.at[g % G], acc.at[idxb.at[psl, pt, 1]],
                               sems, add=True)
    for g in range(M - G, M):
      sc[g].wait()


def _make_scatter(F, total_blocks):
  NZ = ZROWS // CHUNK

  def _writeback(acc, out, rows, sema, semo, s, ci):
    vw = [None, None]
    for q in range(NZ):
      b = q % 2
      if q >= 2:
        vw[b].wait()
      r0 = s * ZROWS + q * CHUNK
      pltpu.async_copy(acc.at[pl.ds(r0, CHUNK)], rows.at[b], sema).wait()
      vw[b] = pltpu.async_copy(rows.at[b], out.at[ci, pl.ds(r0, CHUNK)], semo)
    for d in vw:
      d.wait()

  def body(edges, ta, tb, out, idxb, rows, acc, semi, semg, sems):
    c = lax.axis_index("c")
    s = lax.axis_index("s")
    @pl.loop(0, CHUNK)
    def _z(r):
      rows[0, r, :] = jnp.zeros((F,), jnp.float32)

    zd = [pltpu.async_copy(rows.at[0],
                           acc.at[pl.ds(s * ZROWS + q * CHUNK, CHUNK)], semg)
          for q in range(NZ)]
    for d in zd:
      d.wait()
    plsc.subcore_barrier()
    bpt = total_blocks // NS
    base = s * bpt
    pl.when(c == 0)(lambda: _edge_loop(
        edges, ta, acc, idxb, rows, semi, semg, sems, base, bpt))
    pl.when(c == 1)(lambda: _edge_loop(
        edges, tb, acc, idxb, rows, semi, semg, sems, base, bpt))
    plsc.subcore_barrier()
    pl.when(c == 0)(lambda: _writeback(acc, out, rows, semi, sems, s, 0))
    pl.when(c == 1)(lambda: _writeback(acc, out, rows, semi, sems, s, 1))

  return pl.kernel(
      body,
      out_type=jax.ShapeDtypeStruct((NC, R, F), jnp.float32),
      mesh=_mesh,
      compiler_params=pltpu.CompilerParams(use_tc_tiling_on_sc=False),
      scratch_types=[
          pltpu.VMEM((IB, W, 2, CHUNK), jnp.int32),
          pltpu.VMEM((G, CHUNK, F), jnp.float32),
          pltpu.VMEM_SHARED((R, F), jnp.float32),
          pltpu.SemaphoreType.DMA,
          pltpu.SemaphoreType.DMA,
          pltpu.SemaphoreType.DMA,
      ],
  )


def _make_deg(total_blocks):
  def body(edges, zer, out, idxb, ones_v, acc, semi, sems):
    c = lax.axis_index("c")
    s = lax.axis_index("s")
    sl = pl.ds(s * ZROWS, ZROWS)
    for i in range(CHUNK // 16):
      ones_v[pl.ds(i * 16, 16)] = jnp.ones((16,), jnp.float32)
    pltpu.sync_copy(zer, acc.at[sl])
    plsc.subcore_barrier()
    bpt = total_blocks // (NC * NS)
    base = (s * NC + c) * bpt

    @pl.loop(0, bpt // IB)
    def _outer(i):
      b0 = base + i * IB
      ld = [pltpu.async_copy(edges.at[b0 + sl_], idxb.at[sl_], semi)
            for sl_ in range(IB)]
      sc = []
      for sl_ in range(IB):
        ld[sl_].wait()
        for t in range(W):
          sc.append(pltpu.async_copy(ones_v, acc.at[idxb.at[sl_, t, 1]],
                                     sems, add=True))
      for s_ in sc:
        s_.wait()

    plsc.subcore_barrier()
    pl.when(c == 0)(lambda: pltpu.sync_copy(acc.at[sl], out.at[0, sl]))
    pl.when(c == 1)(lambda: pltpu.sync_copy(acc.at[sl], out.at[1, sl]))

  return pl.kernel(
      body,
      out_type=jax.ShapeDtypeStruct((NC, R), jnp.float32),
      mesh=_mesh,
      compiler_params=pltpu.CompilerParams(use_tc_tiling_on_sc=False),
      scratch_types=[
          pltpu.VMEM((IB, W, 2, CHUNK), jnp.int32),
          pltpu.VMEM((CHUNK,), jnp.float32),
          pltpu.VMEM_SHARED((R,), jnp.float32),
          pltpu.SemaphoreType.DMA,
          pltpu.SemaphoreType.DMA,
      ],
  )


def _mm1_body(x_ref, w_ref, o_ref):
  o_ref[...] = jnp.dot(x_ref[...], w_ref[...],
                       preferred_element_type=jnp.float32)


def _scale1_body(h_ref, dp0_ref, dp1_ref, hsa_ref, hsb_ref, dis_ref):
  dis = lax.rsqrt(dp0_ref[...] + dp1_ref[...] + 1.0)
  hs = h_ref[...] * dis
  hsa_ref[...] = hs[:, :16]
  hsb_ref[...] = hs[:, 16:]
  dis_ref[...] = dis


def _mid_body(a0_ref, a1_ref, hsa_ref, hsb_ref, dis_ref, w2_ref, b1_ref,
              hs2a_ref, hs2b_ref):
  dis = dis_ref[...]
  b1 = b1_ref[...]
  r0 = jnp.maximum((a0_ref[...] + hsa_ref[...]) * dis + b1[:, :16], 0.0)
  r1 = jnp.maximum((a1_ref[...] + hsb_ref[...]) * dis + b1[:, 16:], 0.0)
  w2 = w2_ref[...]
  h2 = (jnp.dot(r0, w2[:16, :], preferred_element_type=jnp.float32)
        + jnp.dot(r1, w2[16:, :], preferred_element_type=jnp.float32))
  hs2 = h2 * dis
  zpad = jnp.zeros((hs2.shape[0], 6), jnp.float32)
  hs2a_ref[...] = jnp.concatenate([hs2[:, :10], zpad], axis=-1)
  hs2b_ref[...] = jnp.concatenate([hs2[:, 10:], zpad], axis=-1)


def _post_body(a0_ref, a1_ref, hs2a_ref, hs2b_ref, dis_ref, b2_ref, o_ref):
  dis = dis_ref[...]
  b2 = b2_ref[...]
  v0 = (a0_ref[...] + hs2a_ref[...])[:, :10] * dis + b2[:, :10]
  v1 = (a1_ref[...] + hs2b_ref[...])[:, :10] * dis + b2[:, 10:]
  o_ref[...] = jnp.concatenate([v0, v1], axis=-1)


def _row_block(F):
  return pl.BlockSpec((BN, F), lambda i: (i, 0))


def _full_block(shape):
  return pl.BlockSpec(shape, lambda i: (0, 0))


def kernel(x, edge_index, W1, b1, W2, b2):
  x = x.astype(jnp.float32)
  ei = edge_index.astype(jnp.int32)
  E = ei.shape[1]
  group = NC * NS * SUP
  E_pad = ((E + group - 1) // group) * group
  pad = E_pad - E
  src = jnp.concatenate([ei[0], jnp.zeros((pad,), jnp.int32)])
  dst = jnp.concatenate([ei[1], jnp.full((pad,), N, jnp.int32)])
  edges = jnp.stack([src.reshape(-1, W, CHUNK), dst.reshape(-1, W, CHUNK)],
                    axis=2)
  total_blocks = E_pad // (W * CHUNK)
  zflat = jnp.zeros((ZROWS,), jnp.float32)

  grid = (N // BN,)

  degp = _make_deg(total_blocks)(edges, zflat)
  h1 = pl.pallas_call(
      _mm1_body, grid=grid,
      in_specs=[_row_block(20), _full_block((20, 32))],
      out_specs=_row_block(32),
      out_shape=jax.ShapeDtypeStruct((N, 32), jnp.float32))(x, W1)

  dp0 = degp[0, :N].reshape(N, 1)
  dp1 = degp[1, :N].reshape(N, 1)
  hsa, hsb, dis = pl.pallas_call(
      _scale1_body, grid=grid,
      in_specs=[_row_block(32), _row_block(1), _row_block(1)],
      out_specs=[_row_block(16), _row_block(16), _row_block(1)],
      out_shape=[jax.ShapeDtypeStruct((N, 16), jnp.float32),
                 jax.ShapeDtypeStruct((N, 16), jnp.float32),
                 jax.ShapeDtypeStruct((N, 1), jnp.float32)])(h1, dp0, dp1)

  agg1 = _make_scatter(16, total_blocks)(edges, hsa, hsb)
  a10 = agg1[0, :N]
  a11 = agg1[1, :N]

  hs2a, hs2b = pl.pallas_call(
      _mid_body, grid=grid,
      in_specs=[_row_block(16), _row_block(16), _row_block(16),
                _row_block(16), _row_block(1), _full_block((32, 20)),
                _full_block((1, 32))],
      out_specs=[_row_block(16), _row_block(16)],
      out_shape=[jax.ShapeDtypeStruct((N, 16), jnp.float32),
                 jax.ShapeDtypeStruct((N, 16), jnp.float32)])(
          a10, a11, hsa, hsb, dis, W2, b1.reshape(1, 32))

  agg2 = _make_scatter(16, total_blocks)(edges, hs2a, hs2b)
  a20 = agg2[0, :N]
  a21 = agg2[1, :N]

  out = pl.pallas_call(
      _post_body, grid=grid,
      in_specs=[_row_block(16), _row_block(16), _row_block(16),
                _row_block(16), _row_block(1), _full_block((1, 20))],
      out_specs=_row_block(20),
      out_shape=jax.ShapeDtypeStruct((N, 20), jnp.float32))(
          a20, a21, hs2a, hs2b, dis, b2.reshape(1, 20))
  return out


# speedup vs baseline: 1.6171x; 1.1308x over previous
import jax
import jax.numpy as jnp
from jax import lax
from jax.experimental import pallas as pl
from jax.experimental.pallas import tpu as pltpu
from jax.experimental.pallas import tpu_sc as plsc

N = 100000
NC, NS = 2, 16
CHUNK = 128
W = 8
IB = 2
G = 8
GD = 6
SUP = IB * W * CHUNK
R = 100352
ZROWS = R // NS
BN = 4000

_mesh = plsc.VectorSubcoreMesh(
    core_axis_name="c", subcore_axis_name="s", num_cores=NC, num_subcores=NS)


def _edge_loop(srcb, dstb, tab, acc, sidxb, didxb, rows, semi, semg, sems,
               blk_base, n_blk):
  M = IB * W

  @pl.loop(0, n_blk // IB)
  def _outer(i):
    b0 = blk_base + i * IB
    ld = [(pltpu.async_copy(srcb.at[b0 + sl], sidxb.at[sl], semi),
           pltpu.async_copy(dstb.at[b0 + sl], didxb.at[sl], semi))
          for sl in range(IB)]
    gl = [None] * M
    sc = [None] * M
    for sl in range(IB):
      ld[sl][0].wait()
      ld[sl][1].wait()
      for t in range(W):
        g = sl * W + t
        if g >= G:
          sc[g - G].wait()
        gl[g] = pltpu.async_copy(tab.at[sidxb.at[sl, t]], rows.at[g % G],
                                 semg)
        if g >= GD:
          gl[g - GD].wait()
          psl, pt = divmod(g - GD, W)
          sc[g - GD] = pltpu.async_copy(rows.at[(g - GD) % G],
                                        acc.at[didxb.at[psl, pt]],
                                        sems, add=True)
    for g in range(M - GD, M):
      gl[g].wait()
      psl, pt = divmod(g, W)
      sc[g] = pltpu.async_copy(rows.at[g % G], acc.at[didxb.at[psl, pt]],
                               sems, add=True)
    for g in range(M - G, M):
      sc[g].wait()


def _make_scatter(F, total_blocks):
  NZ = ZROWS // CHUNK

  def _writeback(acc, out, rows, sema, semo, s, ci):
    vw = [None, None]
    for q in range(NZ):
      b = q % 2
      if q >= 2:
        vw[b].wait()
      r0 = s * ZROWS + q * CHUNK
      pltpu.async_copy(acc.at[pl.ds(r0, CHUNK)], rows.at[b], sema).wait()
      vw[b] = pltpu.async_copy(rows.at[b], out.at[ci, pl.ds(r0, CHUNK)], semo)
    for d in vw:
      d.wait()

  def body(srcb, dstb, ta, tb, out, sidxb, didxb, rows, acc, semi, semg,
           sems):
    c = lax.axis_index("c")
    s = lax.axis_index("s")
    @pl.loop(0, CHUNK)
    def _z(r):
      rows[0, r, :] = jnp.zeros((F,), jnp.float32)

    zd = [pltpu.async_copy(rows.at[0],
                           acc.at[pl.ds(s * ZROWS + q * CHUNK, CHUNK)], semg)
          for q in range(NZ)]
    for d in zd:
      d.wait()
    plsc.subcore_barrier()
    bpt = total_blocks // NS
    base = s * bpt
    pl.when(c == 0)(lambda: _edge_loop(
        srcb, dstb, ta, acc, sidxb, didxb, rows, semi, semg, sems,
        base, bpt))
    pl.when(c == 1)(lambda: _edge_loop(
        srcb, dstb, tb, acc, sidxb, didxb, rows, semi, semg, sems,
        base, bpt))
    plsc.subcore_barrier()
    pl.when(c == 0)(lambda: _writeback(acc, out, rows, semi, sems, s, 0))
    pl.when(c == 1)(lambda: _writeback(acc, out, rows, semi, sems, s, 1))

  return pl.kernel(
      body,
      out_type=jax.ShapeDtypeStruct((NC, R, F), jnp.float32),
      mesh=_mesh,
      compiler_params=pltpu.CompilerParams(use_tc_tiling_on_sc=False),
      scratch_types=[
          pltpu.VMEM((IB, W, CHUNK), jnp.int32),
          pltpu.VMEM((IB, W, CHUNK), jnp.int32),
          pltpu.VMEM((G, CHUNK, F), jnp.float32),
          pltpu.VMEM_SHARED((R, F), jnp.float32),
          pltpu.SemaphoreType.DMA,
          pltpu.SemaphoreType.DMA,
          pltpu.SemaphoreType.DMA,
      ],
  )


def _make_deg(total_blocks):
  def body(dstb, zer, out, idxb, ones_v, acc, semi, sems):
    c = lax.axis_index("c")
    s = lax.axis_index("s")
    sl = pl.ds(s * ZROWS, ZROWS)
    for i in range(CHUNK // 16):
      ones_v[pl.ds(i * 16, 16)] = jnp.ones((16,), jnp.float32)
    pltpu.sync_copy(zer, acc.at[sl])
    plsc.subcore_barrier()
    bpt = total_blocks // (NC * NS)
    base = (s * NC + c) * bpt

    @pl.loop(0, bpt // IB)
    def _outer(i):
      b0 = base + i * IB
      ld = [pltpu.async_copy(dstb.at[b0 + sl_], idxb.at[sl_], semi)
            for sl_ in range(IB)]
      sc = []
      for sl_ in range(IB):
        ld[sl_].wait()
        for t in range(W):
          sc.append(pltpu.async_copy(ones_v, acc.at[idxb.at[sl_, t]],
                                     sems, add=True))
      for s_ in sc:
        s_.wait()

    plsc.subcore_barrier()
    pl.when(c == 0)(lambda: pltpu.sync_copy(acc.at[sl], out.at[0, sl]))
    pl.when(c == 1)(lambda: pltpu.sync_copy(acc.at[sl], out.at[1, sl]))

  return pl.kernel(
      body,
      out_type=jax.ShapeDtypeStruct((NC, R), jnp.float32),
      mesh=_mesh,
      compiler_params=pltpu.CompilerParams(use_tc_tiling_on_sc=False),
      scratch_types=[
          pltpu.VMEM((IB, W, CHUNK), jnp.int32),
          pltpu.VMEM((CHUNK,), jnp.float32),
          pltpu.VMEM_SHARED((R,), jnp.float32),
          pltpu.SemaphoreType.DMA,
          pltpu.SemaphoreType.DMA,
      ],
  )


def _pre_body(x_ref, w_ref, dp_ref, hsa_ref, hsb_ref, dis_ref):
  dp = dp_ref[...]
  dis = lax.rsqrt(dp[:, 0:1] + dp[:, 1:2] + 1.0)
  h = jnp.dot(x_ref[...], w_ref[...], preferred_element_type=jnp.float32)
  hs = h * dis
  hsa_ref[...] = hs[:, :16]
  hsb_ref[...] = hs[:, 16:]
  dis_ref[...] = dis


def _mid_body(a0_ref, a1_ref, hsa_ref, hsb_ref, dis_ref, w2_ref, b1_ref,
              hs2a_ref, hs2b_ref):
  dis = dis_ref[...]
  b1 = b1_ref[...]
  r0 = jnp.maximum((a0_ref[0] + hsa_ref[...]) * dis + b1[:, :16], 0.0)
  r1 = jnp.maximum((a1_ref[0] + hsb_ref[...]) * dis + b1[:, 16:], 0.0)
  w2 = w2_ref[...]
  h2 = (jnp.dot(r0, w2[:16, :], preferred_element_type=jnp.float32)
        + jnp.dot(r1, w2[16:, :], preferred_element_type=jnp.float32))
  hs2 = h2 * dis
  zpad = jnp.zeros((hs2.shape[0], 6), jnp.float32)
  hs2a_ref[...] = jnp.concatenate([hs2[:, :10], zpad], axis=-1)
  hs2b_ref[...] = jnp.concatenate([hs2[:, 10:], zpad], axis=-1)


def _post_body(a0_ref, a1_ref, hs2a_ref, hs2b_ref, dis_ref, b2_ref, o_ref):
  dis = dis_ref[...]
  b2 = b2_ref[...]
  v0 = (a0_ref[0] + hs2a_ref[...])[:, :10] * dis + b2[:, :10]
  v1 = (a1_ref[0] + hs2b_ref[...])[:, :10] * dis + b2[:, 10:]
  o_ref[...] = jnp.concatenate([v0, v1], axis=-1)


def _row_block(F):
  return pl.BlockSpec((BN, F), lambda i: (i, 0))


def _agg_block(F, ci):
  return pl.BlockSpec((1, BN, F), lambda i, ci=ci: (ci, i, 0))


def _full_block(shape):
  return pl.BlockSpec(shape, lambda i: (0, 0))


def kernel(x, edge_index, W1, b1, W2, b2):
  x = x.astype(jnp.float32)
  ei = edge_index.astype(jnp.int32)
  E = ei.shape[1]
  group = NC * NS * SUP
  E_pad = ((E + group - 1) // group) * group
  pad = E_pad - E
  srcb = jnp.pad(ei[0], (0, pad)).reshape(-1, W, CHUNK)
  dstb = jnp.pad(ei[1], (0, pad), constant_values=N).reshape(-1, W, CHUNK)
  total_blocks = E_pad // (W * CHUNK)
  zflat = jnp.zeros((ZROWS,), jnp.float32)

  grid = (N // BN,)

  degp = _make_deg(total_blocks)(dstb, zflat)
  dp = degp[:, :N].T

  hsa, hsb, dis = pl.pallas_call(
      _pre_body, grid=grid,
      in_specs=[_row_block(20), _full_block((20, 32)), _row_block(2)],
      out_specs=[_row_block(16), _row_block(16), _row_block(1)],
      out_shape=[jax.ShapeDtypeStruct((N, 16), jnp.float32),
                 jax.ShapeDtypeStruct((N, 16), jnp.float32),
                 jax.ShapeDtypeStruct((N, 1), jnp.float32)])(x, W1, dp)

  agg1 = _make_scatter(16, total_blocks)(srcb, dstb, hsa, hsb)

  hs2a, hs2b = pl.pallas_call(
      _mid_body, grid=grid,
      in_specs=[_agg_block(16, 0), _agg_block(16, 1), _row_block(16),
                _row_block(16), _row_block(1), _full_block((32, 20)),
                _full_block((1, 32))],
      out_specs=[_row_block(16), _row_block(16)],
      out_shape=[jax.ShapeDtypeStruct((N, 16), jnp.float32),
                 jax.ShapeDtypeStruct((N, 16), jnp.float32)])(
          agg1, agg1, hsa, hsb, dis, W2, b1.reshape(1, 32))

  agg2 = _make_scatter(16, total_blocks)(srcb, dstb, hs2a, hs2b)

  out = pl.pallas_call(
      _post_body, grid=grid,
      in_specs=[_agg_block(16, 0), _agg_block(16, 1), _row_block(16),
                _row_block(16), _row_block(1), _full_block((1, 20))],
      out_specs=_row_block(20),
      out_shape=jax.ShapeDtypeStruct((N, 20), jnp.float32))(
          agg2, agg2, hs2a, hs2b, dis, b2.reshape(1, 20))
  return out
